# Initial kernel scaffold; baseline (speedup 1.0000x reference)
import jax
import jax.numpy as jnp
from jax.experimental import pallas as pl


def kernel(x, conv_w, conv_b, bn_gamma, bn_beta, w_self1, w_neigh1, b1, w_self2, w_neigh2, b2, w_self3, w_neigh3, b3):
    raise NotImplementedError("write your pallas kernel here")


# speedup vs baseline: 11.3999x; 11.3999x over previous
import functools

import jax
import jax.numpy as jnp
from jax import lax
from jax.experimental import pallas as pl
from jax.experimental.pallas import tpu as pltpu
from jax.experimental.pallas import tpu_sc as plsc

IC = 96
OC = 192
K = 8
B = 2
H = 56
W = 56
N = H * W
BN = B * N

_NEG_SLOPE = 0.05
_BIG = 3.0e38


def _leaky(v):
    return jnp.where(v >= 0, v, _NEG_SLOPE * v)


def _prep_body(x_ref, w_ref, cb_ref, g_ref, b_ref, out_ref):
    col = lax.broadcasted_iota(jnp.int32, (1, N), 1) % W
    ys = []
    for bb in range(B):
        x = x_ref[bb]
        acc = jnp.zeros((IC, N), jnp.float32)
        for ki in range(3):
            for kj in range(3):
                s = (ki - 1) * W + (kj - 1)
                if s > 0:
                    xs = jnp.concatenate(
                        [x[:, s:], jnp.zeros((IC, s), jnp.float32)], axis=1)
                elif s < 0:
                    xs = jnp.concatenate(
                        [jnp.zeros((IC, -s), jnp.float32), x[:, :s]], axis=1)
                else:
                    xs = x
                if kj == 2:
                    xs = jnp.where(col < (W - 1), xs, 0.0)
                elif kj == 0:
                    xs = jnp.where(col > 0, xs, 0.0)
                acc = acc + lax.dot(w_ref[ki, kj], xs,
                                    preferred_element_type=jnp.float32)
        ys.append(acc + cb_ref[...])
    total = jnp.float32(B * N)
    ssum = ys[0].sum(axis=1, keepdims=True) + ys[1].sum(axis=1, keepdims=True)
    mean = ssum / total
    d0 = ys[0] - mean
    d1 = ys[1] - mean
    var = ((d0 * d0).sum(axis=1, keepdims=True)
           + (d1 * d1).sum(axis=1, keepdims=True)) / total
    scale = g_ref[...] * lax.rsqrt(var + 1e-5)
    out_ref[0] = _leaky(d0 * scale + b_ref[...])
    out_ref[1] = _leaky(d1 * scale + b_ref[...])


def _prep_call(x3, wt, cb, g, b):
    return pl.pallas_call(
        _prep_body,
        out_shape=jax.ShapeDtypeStruct((B, IC, N), jnp.float32),
    )(x3, wt, cb, g, b)


_BLK = 392


def _knn_body(h_ref, ht_ref, out_ref):
    hb = h_ref[0]
    ht = ht_ref[0]
    sq = jnp.sum(ht * ht, axis=0, keepdims=True)
    m = sq - 2.0 * lax.dot(hb, ht, preferred_element_type=jnp.float32)
    iota_n = lax.broadcasted_iota(jnp.int32, (_BLK, N), 1)
    base = pl.program_id(0) * N
    cols = []
    for _ in range(K):
        mn = jnp.min(m, axis=1, keepdims=True)
        idxk = jnp.min(jnp.where(m == mn, iota_n, jnp.int32(N)),
                       axis=1, keepdims=True)
        cols.append(idxk)
        m = jnp.where(iota_n == idxk, _BIG, m)
    out_ref[0] = jnp.concatenate(cols, axis=1) + base


def _knn_call(h, ht):
    return pl.pallas_call(
        _knn_body,
        grid=(B, N // _BLK),
        in_specs=[
            pl.BlockSpec((1, _BLK, IC), lambda bb, i: (bb, i, 0)),
            pl.BlockSpec((1, IC, N), lambda bb, i: (bb, 0, 0)),
        ],
        out_specs=pl.BlockSpec((1, _BLK, K), lambda bb, i: (bb, i, 0)),
        out_shape=jax.ShapeDtypeStruct((B, N, K), jnp.int32),
    )(h, ht)


_NW = 32
_NPW = 208
_BNP = _NW * _NPW
_G = 16
_R = _NPW // _G


def _gmean_call(hflat, idxpad, c):
    mesh = plsc.VectorSubcoreMesh(core_axis_name="c", subcore_axis_name="s")

    @functools.partial(
        pl.kernel,
        mesh=mesh,
        compiler_params=pltpu.CompilerParams(use_tc_tiling_on_sc=False),
        out_type=jax.ShapeDtypeStruct((_BNP, c), jnp.float32),
        scratch_types=[
            pltpu.VMEM((_G * K,), jnp.int32),
            pltpu.VMEM((_G * K, c), jnp.float32),
            pltpu.VMEM((_NPW, c), jnp.float32),
            pltpu.SemaphoreType.DMA,
        ],
    )
    def gmean(h_hbm, idx_hbm, out_hbm, idx_v, rows_v, acc_v, sem):
        cid = lax.axis_index("c")
        sid = lax.axis_index("s")
        wid = sid * 2 + cid
        for r in range(_R):
            nbase = wid * _NPW + r * _G
            pltpu.sync_copy(idx_hbm.at[pl.ds(nbase * K, _G * K)], idx_v)
            pltpu.async_copy(h_hbm.at[idx_v], rows_v, sem).wait()

            def body(n, carry):
                for j in range(c // 16):
                    a = rows_v[n * K, pl.ds(j * 16, 16)]
                    for kk in range(1, K):
                        a = a + rows_v[n * K + kk, pl.ds(j * 16, 16)]
                    acc_v[r * _G + n, pl.ds(j * 16, 16)] = a
                return carry

            lax.fori_loop(0, _G, body, 0)
        pltpu.sync_copy(acc_v, out_hbm.at[pl.ds(wid * _NPW, _NPW)])

    return gmean(hflat, idxpad)[:BN]


_GBLK = 784


def _gc_body(h_ref, agg_ref, ws_ref, wn_ref, b_ref, out_ref):
    out = (lax.dot(h_ref[...], ws_ref[...], preferred_element_type=jnp.float32)
           + lax.dot(agg_ref[...] * (1.0 / K), wn_ref[...],
                     preferred_element_type=jnp.float32)
           + b_ref[...])
    out_ref[...] = _leaky(out)


def _gc_call(h, agg, ws, wn, b):
    cin = ws.shape[0]
    cout = ws.shape[1]
    return pl.pallas_call(
        _gc_body,
        grid=(BN // _GBLK,),
        in_specs=[
            pl.BlockSpec((_GBLK, cin), lambda i: (i, 0)),
            pl.BlockSpec((_GBLK, cin), lambda i: (i, 0)),
            pl.BlockSpec((cin, cout), lambda i: (0, 0)),
            pl.BlockSpec((cin, cout), lambda i: (0, 0)),
            pl.BlockSpec((1, cout), lambda i: (0, 0)),
        ],
        out_specs=pl.BlockSpec((_GBLK, cout), lambda i: (i, 0)),
        out_shape=jax.ShapeDtypeStruct((BN, cout), jnp.float32),
    )(h, agg, ws, wn, b)


def kernel(x, conv_w, conv_b, bn_gamma, bn_beta, w_self1, w_neigh1, b1,
           w_self2, w_neigh2, b2, w_self3, w_neigh3, b3):
    x3 = x.reshape(B, IC, N)
    wt = conv_w.transpose(2, 3, 0, 1)
    ht = _prep_call(x3, wt, conv_b.reshape(IC, 1), bn_gamma.reshape(IC, 1),
                    bn_beta.reshape(IC, 1))
    h = ht.transpose(0, 2, 1)
    gidx = _knn_call(h, ht)
    idxflat = jnp.pad(gidx.reshape(-1), (0, (_BNP - BN) * K))
    hflat = h.reshape(BN, IC)

    agg1 = _gmean_call(hflat, idxflat, IC)
    h1 = _gc_call(hflat, agg1, w_self1, w_neigh1, b1.reshape(1, IC))
    agg2 = _gmean_call(h1, idxflat, IC)
    h2 = _gc_call(h1, agg2, w_self2, w_neigh2, b2.reshape(1, OC))
    agg3 = _gmean_call(h2, idxflat, OC)
    h3 = _gc_call(h2, agg3, w_self3, w_neigh3, b3.reshape(1, OC))

    return h3.reshape(B, N, OC).transpose(0, 2, 1).reshape(B, OC, H, W)


# speedup vs baseline: 11.7380x; 1.0297x over previous
import functools

import jax
import jax.numpy as jnp
from jax import lax
from jax.experimental import pallas as pl
from jax.experimental.pallas import tpu as pltpu
from jax.experimental.pallas import tpu_sc as plsc

IC = 96
OC = 192
K = 8
B = 2
H = 56
W = 56
N = H * W
BN = B * N

_NEG_SLOPE = 0.05
_BIG = 3.0e38


def _leaky(v):
    return jnp.where(v >= 0, v, _NEG_SLOPE * v)


def _prep_body(x_ref, w_ref, cb_ref, g_ref, b_ref, out_ref):
    col = lax.broadcasted_iota(jnp.int32, (1, N), 1) % W
    ys = []
    for bb in range(B):
        x = x_ref[bb]
        acc = jnp.zeros((IC, N), jnp.float32)
        for ki in range(3):
            for kj in range(3):
                s = (ki - 1) * W + (kj - 1)
                if s > 0:
                    xs = jnp.concatenate(
                        [x[:, s:], jnp.zeros((IC, s), jnp.float32)], axis=1)
                elif s < 0:
                    xs = jnp.concatenate(
                        [jnp.zeros((IC, -s), jnp.float32), x[:, :s]], axis=1)
                else:
                    xs = x
                if kj == 2:
                    xs = jnp.where(col < (W - 1), xs, 0.0)
                elif kj == 0:
                    xs = jnp.where(col > 0, xs, 0.0)
                acc = acc + lax.dot(w_ref[ki, kj], xs,
                                    preferred_element_type=jnp.float32)
        ys.append(acc + cb_ref[...])
    total = jnp.float32(B * N)
    ssum = ys[0].sum(axis=1, keepdims=True) + ys[1].sum(axis=1, keepdims=True)
    mean = ssum / total
    d0 = ys[0] - mean
    d1 = ys[1] - mean
    var = ((d0 * d0).sum(axis=1, keepdims=True)
           + (d1 * d1).sum(axis=1, keepdims=True)) / total
    scale = g_ref[...] * lax.rsqrt(var + 1e-5)
    out_ref[0] = _leaky(d0 * scale + b_ref[...])
    out_ref[1] = _leaky(d1 * scale + b_ref[...])


def _prep_call(x3, wt, cb, g, b):
    return pl.pallas_call(
        _prep_body,
        out_shape=jax.ShapeDtypeStruct((B, IC, N), jnp.float32),
    )(x3, wt, cb, g, b)


_BLK = 392


def _knn_body(h_ref, ht_ref, out_ref):
    hb = h_ref[0]
    ht = ht_ref[0]
    sq = jnp.sum(ht * ht, axis=0, keepdims=True)
    m = sq - 2.0 * lax.dot(hb, ht, preferred_element_type=jnp.float32)
    iota_n = lax.broadcasted_iota(jnp.int32, (_BLK, N), 1)
    base = pl.program_id(0) * N
    cols = []
    for _ in range(K):
        mn = jnp.min(m, axis=1, keepdims=True)
        idxk = jnp.min(jnp.where(m == mn, iota_n, jnp.int32(N)),
                       axis=1, keepdims=True)
        cols.append(idxk)
        m = jnp.where(iota_n == idxk, _BIG, m)
    out_ref[0] = jnp.concatenate(cols, axis=1) + base


def _knn_call(h, ht):
    return pl.pallas_call(
        _knn_body,
        grid=(B, N // _BLK),
        in_specs=[
            pl.BlockSpec((1, _BLK, IC), lambda bb, i: (bb, i, 0)),
            pl.BlockSpec((1, IC, N), lambda bb, i: (bb, 0, 0)),
        ],
        out_specs=pl.BlockSpec((1, _BLK, K), lambda bb, i: (bb, i, 0)),
        out_shape=jax.ShapeDtypeStruct((B, N, K), jnp.int32),
    )(h, ht)


_NW = 32
_NPW = 208
_BNP = _NW * _NPW
_G = 16
_R = _NPW // _G


def _gmean_call(hflat, idx3, c):
    mesh = plsc.VectorSubcoreMesh(core_axis_name="c", subcore_axis_name="s")

    @functools.partial(
        pl.kernel,
        mesh=mesh,
        compiler_params=pltpu.CompilerParams(use_tc_tiling_on_sc=False),
        out_type=jax.ShapeDtypeStruct((_BNP, c), jnp.float32),
        scratch_types=[
            pltpu.VMEM((_R, _G * K), jnp.int32),
            pltpu.VMEM((2, _G * K, c), jnp.float32),
            pltpu.VMEM((_NPW, c), jnp.float32),
            pltpu.SemaphoreType.DMA,
            pltpu.SemaphoreType.DMA,
        ],
    )
    def gmean(h_hbm, idx_hbm, out_hbm, idx_v, rows_v, acc_v, sem0, sem1):
        cid = lax.axis_index("c")
        sid = lax.axis_index("s")
        wid = sid * 2 + cid
        sems = (sem0, sem1)
        pltpu.sync_copy(idx_hbm.at[wid], idx_v)

        def start(r):
            return pltpu.async_copy(h_hbm.at[idx_v.at[r]],
                                    rows_v.at[r % 2], sems[r % 2])

        pending = {0: start(0)}
        for r in range(_R):
            if r + 1 < _R:
                pending[r + 1] = start(r + 1)
            pending.pop(r).wait()
            buf = r % 2

            def body(n, carry):
                for j in range(c // 16):
                    a = rows_v[buf, n * K, pl.ds(j * 16, 16)]
                    for kk in range(1, K):
                        a = a + rows_v[buf, n * K + kk, pl.ds(j * 16, 16)]
                    acc_v[r * _G + n, pl.ds(j * 16, 16)] = a
                return carry

            lax.fori_loop(0, _G, body, 0)
        pltpu.sync_copy(acc_v, out_hbm.at[pl.ds(wid * _NPW, _NPW)])

    return gmean(hflat, idx3)[:BN]


_GBLK = 784


def _gc_body(h_ref, agg_ref, ws_ref, wn_ref, b_ref, out_ref):
    out = (lax.dot(h_ref[...], ws_ref[...], preferred_element_type=jnp.float32)
           + lax.dot(agg_ref[...] * (1.0 / K), wn_ref[...],
                     preferred_element_type=jnp.float32)
           + b_ref[...])
    out_ref[...] = _leaky(out)


def _gc_call(h, agg, ws, wn, b):
    cin = ws.shape[0]
    cout = ws.shape[1]
    return pl.pallas_call(
        _gc_body,
        grid=(BN // _GBLK,),
        in_specs=[
            pl.BlockSpec((_GBLK, cin), lambda i: (i, 0)),
            pl.BlockSpec((_GBLK, cin), lambda i: (i, 0)),
            pl.BlockSpec((cin, cout), lambda i: (0, 0)),
            pl.BlockSpec((cin, cout), lambda i: (0, 0)),
            pl.BlockSpec((1, cout), lambda i: (0, 0)),
        ],
        out_specs=pl.BlockSpec((_GBLK, cout), lambda i: (i, 0)),
        out_shape=jax.ShapeDtypeStruct((BN, cout), jnp.float32),
    )(h, agg, ws, wn, b)


def kernel(x, conv_w, conv_b, bn_gamma, bn_beta, w_self1, w_neigh1, b1,
           w_self2, w_neigh2, b2, w_self3, w_neigh3, b3):
    x3 = x.reshape(B, IC, N)
    wt = conv_w.transpose(2, 3, 0, 1)
    ht = _prep_call(x3, wt, conv_b.reshape(IC, 1), bn_gamma.reshape(IC, 1),
                    bn_beta.reshape(IC, 1))
    h = ht.transpose(0, 2, 1)
    gidx = _knn_call(h, ht)
    idx3 = jnp.pad(gidx.reshape(-1),
                   (0, (_BNP - BN) * K)).reshape(_NW, _R, _G * K)
    hflat = h.reshape(BN, IC)

    agg1 = _gmean_call(hflat, idx3, IC)
    h1 = _gc_call(hflat, agg1, w_self1, w_neigh1, b1.reshape(1, IC))
    agg2 = _gmean_call(h1, idx3, IC)
    h2 = _gc_call(h1, agg2, w_self2, w_neigh2, b2.reshape(1, OC))
    agg3 = _gmean_call(h2, idx3, OC)
    h3 = _gc_call(h2, agg3, w_self3, w_neigh3, b3.reshape(1, OC))

    return h3.reshape(B, N, OC).transpose(0, 2, 1).reshape(B, OC, H, W)


# speedup vs baseline: 12.0932x; 1.0303x over previous
import functools

import jax
import jax.numpy as jnp
from jax import lax
from jax.experimental import pallas as pl
from jax.experimental.pallas import tpu as pltpu
from jax.experimental.pallas import tpu_sc as plsc

IC = 96
OC = 192
K = 8
B = 2
H = 56
W = 56
N = H * W
BN = B * N

_NEG_SLOPE = 0.05
_BIG = 3.0e38


def _leaky(v):
    return jnp.where(v >= 0, v, _NEG_SLOPE * v)


def _prep_body(x_ref, w_ref, cb_ref, g_ref, b_ref, out_ref):
    col = lax.broadcasted_iota(jnp.int32, (1, N), 1) % W
    ys = []
    for bb in range(B):
        x = x_ref[bb]
        acc = jnp.zeros((IC, N), jnp.float32)
        for ki in range(3):
            for kj in range(3):
                s = (ki - 1) * W + (kj - 1)
                if s > 0:
                    xs = jnp.concatenate(
                        [x[:, s:], jnp.zeros((IC, s), jnp.float32)], axis=1)
                elif s < 0:
                    xs = jnp.concatenate(
                        [jnp.zeros((IC, -s), jnp.float32), x[:, :s]], axis=1)
                else:
                    xs = x
                if kj == 2:
                    xs = jnp.where(col < (W - 1), xs, 0.0)
                elif kj == 0:
                    xs = jnp.where(col > 0, xs, 0.0)
                acc = acc + lax.dot(w_ref[ki, kj], xs,
                                    preferred_element_type=jnp.float32)
        ys.append(acc + cb_ref[...])
    total = jnp.float32(B * N)
    ssum = ys[0].sum(axis=1, keepdims=True) + ys[1].sum(axis=1, keepdims=True)
    mean = ssum / total
    d0 = ys[0] - mean
    d1 = ys[1] - mean
    var = ((d0 * d0).sum(axis=1, keepdims=True)
           + (d1 * d1).sum(axis=1, keepdims=True)) / total
    scale = g_ref[...] * lax.rsqrt(var + 1e-5)
    out_ref[0] = _leaky(d0 * scale + b_ref[...])
    out_ref[1] = _leaky(d1 * scale + b_ref[...])


def _prep_call(x3, wt, cb, g, b):
    return pl.pallas_call(
        _prep_body,
        out_shape=jax.ShapeDtypeStruct((B, IC, N), jnp.float32),
    )(x3, wt, cb, g, b)


_BLK = 392


def _knn_body(h_ref, ht_ref, out_ref):
    hb = h_ref[0]
    ht = ht_ref[0]
    sq = jnp.sum(ht * ht, axis=0, keepdims=True)
    m = sq - 2.0 * lax.dot(hb, ht, preferred_element_type=jnp.float32)
    iota_n = lax.broadcasted_iota(jnp.int32, (_BLK, N), 1)
    base = pl.program_id(0) * N
    cols = []
    for _ in range(K):
        mn = jnp.min(m, axis=1, keepdims=True)
        idxk = jnp.min(jnp.where(m == mn, iota_n, jnp.int32(N)),
                       axis=1, keepdims=True)
        cols.append(idxk)
        m = jnp.where(iota_n == idxk, _BIG, m)
    out_ref[0] = jnp.concatenate(cols, axis=1) + base


def _knn_call(h, ht):
    return pl.pallas_call(
        _knn_body,
        grid=(B, N // _BLK),
        in_specs=[
            pl.BlockSpec((1, _BLK, IC), lambda bb, i: (bb, i, 0)),
            pl.BlockSpec((1, IC, N), lambda bb, i: (bb, 0, 0)),
        ],
        out_specs=pl.BlockSpec((1, _BLK, K), lambda bb, i: (bb, i, 0)),
        out_shape=jax.ShapeDtypeStruct((B, N, K), jnp.int32),
    )(h, ht)


_NW = 32
_NPW = 208
_BNP = _NW * _NPW
_CH = 104


def _gmean_call(hflat, idx4, c):
    mesh = plsc.VectorSubcoreMesh(core_axis_name="c", subcore_axis_name="s")

    @functools.partial(
        pl.kernel,
        mesh=mesh,
        compiler_params=pltpu.CompilerParams(use_tc_tiling_on_sc=False),
        out_type=jax.ShapeDtypeStruct((_BNP, c), jnp.float32),
        scratch_types=[
            pltpu.VMEM((K, _NPW // _CH, _CH), jnp.int32),
            pltpu.VMEM((_NPW, c), jnp.float32),
            pltpu.SemaphoreType.DMA,
        ],
    )
    def gmean(h_hbm, idx_hbm, out_hbm, idx_v, acc_v, sem):
        cid = lax.axis_index("c")
        sid = lax.axis_index("s")
        wid = sid * 2 + cid
        pltpu.sync_copy(idx_hbm.at[wid], idx_v)

        def zbody(i, carry):
            for j in range(c // 16):
                acc_v[i, pl.ds(j * 16, 16)] = jnp.zeros((16,), jnp.float32)
            return carry

        lax.fori_loop(0, _NPW, zbody, 0)
        cps = []
        for k in range(K):
            for ch in range(_NPW // _CH):
                cps.append(pltpu.async_copy(
                    h_hbm.at[idx_v.at[k, ch]],
                    acc_v.at[pl.ds(ch * _CH, _CH)],
                    sem, add=True))
        for cp in cps:
            cp.wait()
        pltpu.sync_copy(acc_v, out_hbm.at[pl.ds(wid * _NPW, _NPW)])

    return gmean(hflat, idx4)[:BN]


_GBLK = 784


def _gc_body(h_ref, agg_ref, ws_ref, wn_ref, b_ref, out_ref):
    out = (lax.dot(h_ref[...], ws_ref[...], preferred_element_type=jnp.float32)
           + lax.dot(agg_ref[...] * (1.0 / K), wn_ref[...],
                     preferred_element_type=jnp.float32)
           + b_ref[...])
    out_ref[...] = _leaky(out)


def _gc_call(h, agg, ws, wn, b):
    cin = ws.shape[0]
    cout = ws.shape[1]
    return pl.pallas_call(
        _gc_body,
        grid=(BN // _GBLK,),
        in_specs=[
            pl.BlockSpec((_GBLK, cin), lambda i: (i, 0)),
            pl.BlockSpec((_GBLK, cin), lambda i: (i, 0)),
            pl.BlockSpec((cin, cout), lambda i: (0, 0)),
            pl.BlockSpec((cin, cout), lambda i: (0, 0)),
            pl.BlockSpec((1, cout), lambda i: (0, 0)),
        ],
        out_specs=pl.BlockSpec((_GBLK, cout), lambda i: (i, 0)),
        out_shape=jax.ShapeDtypeStruct((BN, cout), jnp.float32),
    )(h, agg, ws, wn, b)


def kernel(x, conv_w, conv_b, bn_gamma, bn_beta, w_self1, w_neigh1, b1,
           w_self2, w_neigh2, b2, w_self3, w_neigh3, b3):
    x3 = x.reshape(B, IC, N)
    wt = conv_w.transpose(2, 3, 0, 1)
    ht = _prep_call(x3, wt, conv_b.reshape(IC, 1), bn_gamma.reshape(IC, 1),
                    bn_beta.reshape(IC, 1))
    h = ht.transpose(0, 2, 1)
    gidx = _knn_call(h, ht)
    idx4 = jnp.pad(gidx.reshape(BN, K), ((0, _BNP - BN), (0, 0))).reshape(
        _NW, _NPW // _CH, _CH, K).transpose(0, 3, 1, 2)
    hflat = h.reshape(BN, IC)

    agg1 = _gmean_call(hflat, idx4, IC)
    h1 = _gc_call(hflat, agg1, w_self1, w_neigh1, b1.reshape(1, IC))
    agg2 = _gmean_call(h1, idx4, IC)
    h2 = _gc_call(h1, agg2, w_self2, w_neigh2, b2.reshape(1, OC))
    agg3 = _gmean_call(h2, idx4, OC)
    h3 = _gc_call(h2, agg3, w_self3, w_neigh3, b3.reshape(1, OC))

    return h3.reshape(B, N, OC).transpose(0, 2, 1).reshape(B, OC, H, W)
